# Initial kernel scaffold; baseline (speedup 1.0000x reference)
import jax
import jax.numpy as jnp
from jax.experimental import pallas as pl


def kernel(x, edge_index, batch, W1, b1, W2, b2, W3, b3):
    raise NotImplementedError("write your pallas kernel here")


# speedup vs baseline: 21.5842x; 21.5842x over previous
import functools

import jax
import jax.numpy as jnp
from jax import lax
from jax.experimental import pallas as pl
from jax.experimental.pallas import tpu as pltpu
from jax.experimental.pallas import tpu_sc as plsc

N = 10000
E = 320000
D = 128
G = 128
NC, NS = 2, 16
NW = NC * NS
CHUNK = 128
NCHUNK = 80
EPW = NCHUNK * CHUNK
PE = NW * EPW
NPAD = 10240
RPT = NPAD // NS

_mesh = plsc.VectorSubcoreMesh(
    core_axis_name="c", subcore_axis_name="s", num_cores=NC, num_subcores=NS)


@functools.partial(
    pl.kernel,
    out_type=jax.ShapeDtypeStruct((NC, NPAD), jnp.float32),
    mesh=_mesh,
    scratch_types=[
        pltpu.VMEM((NCHUNK, CHUNK), jnp.int32),
        pltpu.VMEM((CHUNK,), jnp.float32),
        pltpu.VMEM((NPAD // NS,), jnp.float32),
        pltpu.VMEM_SHARED((NPAD,), jnp.float32),
    ],
)
def _deg_k(dst_hbm, out_hbm, dst_v, ones_v, stage_v, hist_sh):
    c = lax.axis_index("c")
    s = lax.axis_index("s")
    w = c * NS + s
    pltpu.sync_copy(dst_hbm.at[w], dst_v)
    for i in range(CHUNK // 16):
        ones_v[pl.ds(i * 16, 16)] = jnp.ones((16,), jnp.float32)
    zeros = jnp.zeros((16,), jnp.float32)
    rpt = NPAD // NS

    def zbody(i, carry):
        stage_v[pl.ds(i * 16, 16)] = zeros
        return carry

    lax.fori_loop(0, rpt // 16, zbody, 0)
    pltpu.sync_copy(stage_v, hist_sh.at[pl.ds(s * rpt, rpt)])
    plsc.subcore_barrier()

    @pl.loop(0, NCHUNK)
    def _(j):
        pltpu.sync_copy(ones_v, hist_sh.at[dst_v.at[j]], add=True)

    plsc.subcore_barrier()
    pltpu.sync_copy(hist_sh.at[pl.ds(s * rpt, rpt)], stage_v)
    pltpu.sync_copy(stage_v, out_hbm.at[c, pl.ds(s * rpt, rpt)])


@functools.partial(
    pl.kernel,
    out_type=jax.ShapeDtypeStruct((NC, NPAD, D), jnp.float32),
    mesh=_mesh,
    scratch_types=[
        pltpu.VMEM((NCHUNK, CHUNK), jnp.int32),
        pltpu.VMEM((NCHUNK, CHUNK), jnp.int32),
        pltpu.VMEM((CHUNK, D), jnp.float32),
        pltpu.VMEM_SHARED((NPAD, D), jnp.float32),
        pltpu.SemaphoreType.DMA,
    ],
)
def _spmm_k(y_hbm, src_hbm, dst_hbm, zero_hbm, out_hbm,
            src_v, dst_v, rows_v, acc_sh, sem0):
    c = lax.axis_index("c")
    s = lax.axis_index("s")
    w = c * NS + s
    pltpu.sync_copy(src_hbm.at[w], src_v)
    pltpu.sync_copy(dst_hbm.at[w], dst_v)
    pltpu.sync_copy(zero_hbm, rows_v)
    for k in range(RPT // CHUNK):
        pltpu.sync_copy(rows_v,
                        acc_sh.at[pl.ds(s * RPT + k * CHUNK, CHUNK)])
    plsc.subcore_barrier()

    @pl.loop(0, NCHUNK)
    def _(j):
        pltpu.async_copy(y_hbm.at[src_v.at[j]], rows_v, sem0).wait()
        pltpu.sync_copy(rows_v, acc_sh.at[dst_v.at[j]], add=True)

    plsc.subcore_barrier()
    for k in range(RPT // CHUNK):
        r0 = s * RPT + k * CHUNK
        pltpu.sync_copy(acc_sh.at[pl.ds(r0, CHUNK)], rows_v)
        pltpu.sync_copy(rows_v, out_hbm.at[c, pl.ds(r0, CHUNK)])


BLK = 2000


def _tc1(x, W1, deg_part_t):
    def body(x_ref, w_ref, dp_ref, y_ref, dinv_ref):
        deg = jnp.sum(dp_ref[...], axis=1) + 1.0
        dinv = lax.rsqrt(deg)[:, None]
        y_ref[...] = jnp.dot(x_ref[...], w_ref[...],
                             preferred_element_type=jnp.float32) * dinv
        dinv_ref[...] = dinv

    return pl.pallas_call(
        body,
        grid=(N // BLK,),
        in_specs=[
            pl.BlockSpec((BLK, D), lambda i: (i, 0)),
            pl.BlockSpec((D, D), lambda i: (0, 0)),
            pl.BlockSpec((BLK, NC), lambda i: (i, 0)),
        ],
        out_specs=[
            pl.BlockSpec((BLK, D), lambda i: (i, 0)),
            pl.BlockSpec((BLK, 1), lambda i: (i, 0)),
        ],
        out_shape=[
            jax.ShapeDtypeStruct((N, D), jnp.float32),
            jax.ShapeDtypeStruct((N, 1), jnp.float32),
        ],
    )(x, W1, deg_part_t)


def _tc2(p, y1, dinv, b1):
    def body(p_ref, y_ref, dinv_ref, b_ref, o_ref):
        t = p_ref[0] + p_ref[1] + y_ref[...]
        h = jnp.maximum(t * dinv_ref[...] + b_ref[...], 0.0)
        o_ref[...] = h * dinv_ref[...]

    return pl.pallas_call(
        body,
        grid=(N // BLK,),
        in_specs=[
            pl.BlockSpec((NC, BLK, D), lambda i: (0, i, 0)),
            pl.BlockSpec((BLK, D), lambda i: (i, 0)),
            pl.BlockSpec((BLK, 1), lambda i: (i, 0)),
            pl.BlockSpec((1, D), lambda i: (0, 0)),
        ],
        out_specs=pl.BlockSpec((BLK, D), lambda i: (i, 0)),
        out_shape=jax.ShapeDtypeStruct((N, D), jnp.float32),
    )(p, y1, dinv, b1)


def _tc3(q, y2, dinv, W2, b2, W3):
    def body(q_ref, y_ref, dinv_ref, w2_ref, b2_ref, w3_ref, o_ref):
        a1 = (q_ref[0] + q_ref[1] + y_ref[...]) * dinv_ref[...]
        h2 = jnp.maximum(
            jnp.dot(a1, w2_ref[...], preferred_element_type=jnp.float32)
            + b2_ref[...], 0.0)
        o_ref[...] = jnp.dot(h2, w3_ref[...],
                             preferred_element_type=jnp.float32) * dinv_ref[...]

    return pl.pallas_call(
        body,
        grid=(N // BLK,),
        in_specs=[
            pl.BlockSpec((NC, BLK, D), lambda i: (0, i, 0)),
            pl.BlockSpec((BLK, D), lambda i: (i, 0)),
            pl.BlockSpec((BLK, 1), lambda i: (i, 0)),
            pl.BlockSpec((D, 2 * D), lambda i: (0, 0)),
            pl.BlockSpec((1, 2 * D), lambda i: (0, 0)),
            pl.BlockSpec((2 * D, D), lambda i: (0, 0)),
        ],
        out_specs=pl.BlockSpec((BLK, D), lambda i: (i, 0)),
        out_shape=jax.ShapeDtypeStruct((N, D), jnp.float32),
    )(q, y2, dinv, W2, b2, W3)


def _tc4(r, y3, dinv, b3, batch2d):
    def body(r_ref, y_ref, dinv_ref, b_ref, batch_ref, o_ref, sums_sc, cnts_sc):
        j = pl.program_id(0)

        @pl.when(j == 0)
        def _():
            sums_sc[...] = jnp.zeros_like(sums_sc)
            cnts_sc[...] = jnp.zeros_like(cnts_sc)

        h3 = (r_ref[0] + r_ref[1] + y_ref[...]) * dinv_ref[...] + b_ref[...]
        gids = lax.broadcasted_iota(jnp.int32, (1, G), 1)
        M = (batch_ref[...] == gids).astype(jnp.float32)
        sums_sc[...] += lax.dot_general(
            M, h3, (((0,), (0,)), ((), ())),
            preferred_element_type=jnp.float32)
        cnts_sc[...] += lax.dot_general(
            M, jnp.ones((BLK, 1), jnp.float32), (((0,), (0,)), ((), ())),
            preferred_element_type=jnp.float32)

        @pl.when(j == pl.num_programs(0) - 1)
        def _():
            o_ref[...] = sums_sc[...] / jnp.maximum(cnts_sc[...], 1.0)

    return pl.pallas_call(
        body,
        grid=(N // BLK,),
        in_specs=[
            pl.BlockSpec((NC, BLK, D), lambda i: (0, i, 0)),
            pl.BlockSpec((BLK, D), lambda i: (i, 0)),
            pl.BlockSpec((BLK, 1), lambda i: (i, 0)),
            pl.BlockSpec((1, D), lambda i: (0, 0)),
            pl.BlockSpec((BLK, 1), lambda i: (i, 0)),
        ],
        out_specs=pl.BlockSpec((G, D), lambda i: (0, 0)),
        out_shape=jax.ShapeDtypeStruct((G, D), jnp.float32),
        scratch_shapes=[
            pltpu.VMEM((G, D), jnp.float32),
            pltpu.VMEM((G, 1), jnp.float32),
        ],
    )(r, y3, dinv, b3, batch2d)


def kernel(x, edge_index, batch, W1, b1, W2, b2, W3, b3):
    src = edge_index[0]
    dst = edge_index[1]
    pad = PE - E
    ar = jnp.arange(pad, dtype=jnp.int32)
    src_p = jnp.concatenate([src, ar % N]).reshape(NW, NCHUNK, CHUNK)
    dst_p = jnp.concatenate([dst, N + ar % (NPAD - N)]).reshape(NW, NCHUNK, CHUNK)

    deg_part = _deg_k(dst_p)
    y1, dinv = _tc1(x, W1, deg_part.T)

    zero_blk = jnp.zeros((CHUNK, D), jnp.float32)
    p = _spmm_k(y1, src_p, dst_p, zero_blk)
    y2 = _tc2(p, y1, dinv, b1.reshape(1, D))
    q = _spmm_k(y2, src_p, dst_p, zero_blk)
    y3 = _tc3(q, y2, dinv, W2, b2.reshape(1, 2 * D), W3)
    r = _spmm_k(y3, src_p, dst_p, zero_blk)
    return _tc4(r, y3, dinv, b3.reshape(1, D), batch.reshape(N, 1))


# speedup vs baseline: 32.3205x; 1.4974x over previous
import functools

import jax
import jax.numpy as jnp
from jax import lax
from jax.experimental import pallas as pl
from jax.experimental.pallas import tpu as pltpu
from jax.experimental.pallas import tpu_sc as plsc

N = 10000
E = 320000
D = 128
G = 128
NC, NS = 2, 16
NW = NC * NS
CHUNK = 128
NCHUNK = 80
EPW = NCHUNK * CHUNK
PE = NW * EPW
NPAD = 10240
RPT = NPAD // NS

_mesh = plsc.VectorSubcoreMesh(
    core_axis_name="c", subcore_axis_name="s", num_cores=NC, num_subcores=NS)


@functools.partial(
    pl.kernel,
    out_type=jax.ShapeDtypeStruct((NC, NPAD), jnp.float32),
    mesh=_mesh,
    scratch_types=[
        pltpu.VMEM((NCHUNK, CHUNK), jnp.int32),
        pltpu.VMEM((CHUNK,), jnp.float32),
        pltpu.VMEM((NPAD // NS,), jnp.float32),
        pltpu.VMEM_SHARED((NPAD,), jnp.float32),
    ],
)
def _deg_k(dst_hbm, out_hbm, dst_v, ones_v, stage_v, hist_sh):
    c = lax.axis_index("c")
    s = lax.axis_index("s")
    w = c * NS + s
    pltpu.sync_copy(dst_hbm.at[w], dst_v)
    for i in range(CHUNK // 16):
        ones_v[pl.ds(i * 16, 16)] = jnp.ones((16,), jnp.float32)
    zeros = jnp.zeros((16,), jnp.float32)
    rpt = NPAD // NS

    def zbody(i, carry):
        stage_v[pl.ds(i * 16, 16)] = zeros
        return carry

    lax.fori_loop(0, rpt // 16, zbody, 0)
    pltpu.sync_copy(stage_v, hist_sh.at[pl.ds(s * rpt, rpt)])
    plsc.subcore_barrier()

    @pl.loop(0, NCHUNK)
    def _(j):
        pltpu.sync_copy(ones_v, hist_sh.at[dst_v.at[j]], add=True)

    plsc.subcore_barrier()
    pltpu.sync_copy(hist_sh.at[pl.ds(s * rpt, rpt)], stage_v)
    pltpu.sync_copy(stage_v, out_hbm.at[c, pl.ds(s * rpt, rpt)])


@functools.partial(
    pl.kernel,
    out_type=jax.ShapeDtypeStruct((NC, NPAD, D), jnp.float32),
    mesh=_mesh,
    scratch_types=[
        pltpu.VMEM((NCHUNK, CHUNK), jnp.int32),
        pltpu.VMEM((2, CHUNK), jnp.int32),
        pltpu.VMEM((2, CHUNK, D), jnp.float32),
        pltpu.VMEM_SHARED((NPAD, D), jnp.float32),
        pltpu.SemaphoreType.DMA,
        pltpu.SemaphoreType.DMA,
        pltpu.SemaphoreType.DMA,
        pltpu.SemaphoreType.DMA,
    ],
)
def _spmm_k(y_hbm, src_hbm, dst_hbm, zero_hbm, out_hbm,
            dst_v, ring_v, rows_v, acc_sh, sg0, sg1, si0, si1):
    c = lax.axis_index("c")
    s = lax.axis_index("s")
    w = c * NS + s
    sgs = (sg0, sg1)
    sis = (si0, si1)
    pltpu.sync_copy(dst_hbm.at[w], dst_v)
    pltpu.sync_copy(zero_hbm, rows_v.at[0])
    for k in range(RPT // CHUNK):
        pltpu.sync_copy(rows_v.at[0],
                        acc_sh.at[pl.ds(s * RPT + k * CHUNK, CHUNK)])
    plsc.subcore_barrier()

    for b in range(2):
        pltpu.async_copy(src_hbm.at[w, b], ring_v.at[b], sis[b])
    for b in range(2):
        pltpu.make_async_copy(src_hbm.at[w, 0], ring_v.at[b], sis[b]).wait()
        pltpu.async_copy(y_hbm.at[ring_v.at[b]], rows_v.at[b], sgs[b])

    @pl.loop(0, NCHUNK, step=2)
    def _(j0):
        for b in range(2):
            j = j0 + b
            pltpu.make_async_copy(
                y_hbm.at[ring_v.at[b]], rows_v.at[b], sgs[b]).wait()

            @pl.when(j + 2 < NCHUNK)
            def _():
                pltpu.async_copy(src_hbm.at[w, j + 2], ring_v.at[b], sis[b])

            pltpu.sync_copy(rows_v.at[b], acc_sh.at[dst_v.at[j]], add=True)

            @pl.when(j + 2 < NCHUNK)
            def _():
                pltpu.make_async_copy(
                    src_hbm.at[w, 0], ring_v.at[b], sis[b]).wait()
                pltpu.async_copy(y_hbm.at[ring_v.at[b]], rows_v.at[b], sgs[b])

    plsc.subcore_barrier()
    for k in range(RPT // CHUNK):
        r0 = s * RPT + k * CHUNK
        pltpu.sync_copy(acc_sh.at[pl.ds(r0, CHUNK)], rows_v.at[0])
        pltpu.sync_copy(rows_v.at[0], out_hbm.at[c, pl.ds(r0, CHUNK)])


BLK = 2000


def _tc1(x, W1, deg_part_t):
    def body(x_ref, w_ref, dp_ref, y_ref, dinv_ref):
        deg = jnp.sum(dp_ref[...], axis=1) + 1.0
        dinv = lax.rsqrt(deg)[:, None]
        y_ref[...] = jnp.dot(x_ref[...], w_ref[...],
                             preferred_element_type=jnp.float32) * dinv
        dinv_ref[...] = dinv

    return pl.pallas_call(
        body,
        grid=(N // BLK,),
        in_specs=[
            pl.BlockSpec((BLK, D), lambda i: (i, 0)),
            pl.BlockSpec((D, D), lambda i: (0, 0)),
            pl.BlockSpec((BLK, NC), lambda i: (i, 0)),
        ],
        out_specs=[
            pl.BlockSpec((BLK, D), lambda i: (i, 0)),
            pl.BlockSpec((BLK, 1), lambda i: (i, 0)),
        ],
        out_shape=[
            jax.ShapeDtypeStruct((N, D), jnp.float32),
            jax.ShapeDtypeStruct((N, 1), jnp.float32),
        ],
    )(x, W1, deg_part_t)


def _tc2(p, y1, dinv, b1):
    def body(p_ref, y_ref, dinv_ref, b_ref, o_ref):
        t = p_ref[0] + p_ref[1] + y_ref[...]
        h = jnp.maximum(t * dinv_ref[...] + b_ref[...], 0.0)
        o_ref[...] = h * dinv_ref[...]

    return pl.pallas_call(
        body,
        grid=(N // BLK,),
        in_specs=[
            pl.BlockSpec((NC, BLK, D), lambda i: (0, i, 0)),
            pl.BlockSpec((BLK, D), lambda i: (i, 0)),
            pl.BlockSpec((BLK, 1), lambda i: (i, 0)),
            pl.BlockSpec((1, D), lambda i: (0, 0)),
        ],
        out_specs=pl.BlockSpec((BLK, D), lambda i: (i, 0)),
        out_shape=jax.ShapeDtypeStruct((N, D), jnp.float32),
    )(p, y1, dinv, b1)


def _tc3(q, y2, dinv, W2, b2, W3):
    def body(q_ref, y_ref, dinv_ref, w2_ref, b2_ref, w3_ref, o_ref):
        a1 = (q_ref[0] + q_ref[1] + y_ref[...]) * dinv_ref[...]
        h2 = jnp.maximum(
            jnp.dot(a1, w2_ref[...], preferred_element_type=jnp.float32)
            + b2_ref[...], 0.0)
        o_ref[...] = jnp.dot(h2, w3_ref[...],
                             preferred_element_type=jnp.float32) * dinv_ref[...]

    return pl.pallas_call(
        body,
        grid=(N // BLK,),
        in_specs=[
            pl.BlockSpec((NC, BLK, D), lambda i: (0, i, 0)),
            pl.BlockSpec((BLK, D), lambda i: (i, 0)),
            pl.BlockSpec((BLK, 1), lambda i: (i, 0)),
            pl.BlockSpec((D, 2 * D), lambda i: (0, 0)),
            pl.BlockSpec((1, 2 * D), lambda i: (0, 0)),
            pl.BlockSpec((2 * D, D), lambda i: (0, 0)),
        ],
        out_specs=pl.BlockSpec((BLK, D), lambda i: (i, 0)),
        out_shape=jax.ShapeDtypeStruct((N, D), jnp.float32),
    )(q, y2, dinv, W2, b2, W3)


def _tc4(r, y3, dinv, b3, batch2d):
    def body(r_ref, y_ref, dinv_ref, b_ref, batch_ref, o_ref, sums_sc, cnts_sc):
        j = pl.program_id(0)

        @pl.when(j == 0)
        def _():
            sums_sc[...] = jnp.zeros_like(sums_sc)
            cnts_sc[...] = jnp.zeros_like(cnts_sc)

        h3 = (r_ref[0] + r_ref[1] + y_ref[...]) * dinv_ref[...] + b_ref[...]
        gids = lax.broadcasted_iota(jnp.int32, (1, G), 1)
        M = (batch_ref[...] == gids).astype(jnp.float32)
        sums_sc[...] += lax.dot_general(
            M, h3, (((0,), (0,)), ((), ())),
            preferred_element_type=jnp.float32)
        cnts_sc[...] += lax.dot_general(
            M, jnp.ones((BLK, 1), jnp.float32), (((0,), (0,)), ((), ())),
            preferred_element_type=jnp.float32)

        @pl.when(j == pl.num_programs(0) - 1)
        def _():
            o_ref[...] = sums_sc[...] / jnp.maximum(cnts_sc[...], 1.0)

    return pl.pallas_call(
        body,
        grid=(N // BLK,),
        in_specs=[
            pl.BlockSpec((NC, BLK, D), lambda i: (0, i, 0)),
            pl.BlockSpec((BLK, D), lambda i: (i, 0)),
            pl.BlockSpec((BLK, 1), lambda i: (i, 0)),
            pl.BlockSpec((1, D), lambda i: (0, 0)),
            pl.BlockSpec((BLK, 1), lambda i: (i, 0)),
        ],
        out_specs=pl.BlockSpec((G, D), lambda i: (0, 0)),
        out_shape=jax.ShapeDtypeStruct((G, D), jnp.float32),
        scratch_shapes=[
            pltpu.VMEM((G, D), jnp.float32),
            pltpu.VMEM((G, 1), jnp.float32),
        ],
    )(r, y3, dinv, b3, batch2d)


def kernel(x, edge_index, batch, W1, b1, W2, b2, W3, b3):
    src = edge_index[0]
    dst = edge_index[1]
    pad = PE - E
    ar = jnp.arange(pad, dtype=jnp.int32)
    src_p = jnp.concatenate([src, ar % N]).reshape(NW, NCHUNK, CHUNK)
    dst_p = jnp.concatenate([dst, N + ar % (NPAD - N)]).reshape(NW, NCHUNK, CHUNK)

    deg_part = _deg_k(dst_p)
    y1, dinv = _tc1(x, W1, deg_part.T)

    zero_blk = jnp.zeros((CHUNK, D), jnp.float32)
    p = _spmm_k(y1, src_p, dst_p, zero_blk)
    y2 = _tc2(p, y1, dinv, b1.reshape(1, D))
    q = _spmm_k(y2, src_p, dst_p, zero_blk)
    y3 = _tc3(q, y2, dinv, W2, b2.reshape(1, 2 * D), W3)
    r = _spmm_k(y3, src_p, dst_p, zero_blk)
    return _tc4(r, y3, dinv, b3.reshape(1, D), batch.reshape(N, 1))


# speedup vs baseline: 32.5435x; 1.0069x over previous
import functools

import jax
import jax.numpy as jnp
from jax import lax
from jax.experimental import pallas as pl
from jax.experimental.pallas import tpu as pltpu
from jax.experimental.pallas import tpu_sc as plsc

N = 10000
E = 320000
D = 128
G = 128
NC, NS = 2, 16
NW = NC * NS
CHUNK = 128
NCHUNK = 80
EPW = NCHUNK * CHUNK
PE = NW * EPW
NPAD = 10240
RPT = NPAD // NS

_mesh = plsc.VectorSubcoreMesh(
    core_axis_name="c", subcore_axis_name="s", num_cores=NC, num_subcores=NS)


@functools.partial(
    pl.kernel,
    out_type=jax.ShapeDtypeStruct((NC, NPAD), jnp.float32),
    mesh=_mesh,
    scratch_types=[
        pltpu.VMEM((NCHUNK, CHUNK), jnp.int32),
        pltpu.VMEM((CHUNK,), jnp.float32),
        pltpu.VMEM((NPAD // NS,), jnp.float32),
        pltpu.VMEM_SHARED((NPAD,), jnp.float32),
    ],
)
def _deg_k(dst_hbm, out_hbm, dst_v, ones_v, stage_v, hist_sh):
    c = lax.axis_index("c")
    s = lax.axis_index("s")
    w = c * NS + s
    pltpu.sync_copy(dst_hbm.at[w], dst_v)
    for i in range(CHUNK // 16):
        ones_v[pl.ds(i * 16, 16)] = jnp.ones((16,), jnp.float32)
    zeros = jnp.zeros((16,), jnp.float32)
    rpt = NPAD // NS

    def zbody(i, carry):
        stage_v[pl.ds(i * 16, 16)] = zeros
        return carry

    lax.fori_loop(0, rpt // 16, zbody, 0)
    pltpu.sync_copy(stage_v, hist_sh.at[pl.ds(s * rpt, rpt)])
    plsc.subcore_barrier()

    @pl.loop(0, NCHUNK)
    def _(j):
        pltpu.sync_copy(ones_v, hist_sh.at[dst_v.at[j]], add=True)

    plsc.subcore_barrier()
    pltpu.sync_copy(hist_sh.at[pl.ds(s * rpt, rpt)], stage_v)
    pltpu.sync_copy(stage_v, out_hbm.at[c, pl.ds(s * rpt, rpt)])


@functools.partial(
    pl.kernel,
    out_type=jax.ShapeDtypeStruct((NC, NPAD, D), jnp.float32),
    mesh=_mesh,
    scratch_types=[
        pltpu.VMEM((NCHUNK, CHUNK), jnp.int32),
        pltpu.VMEM((2, CHUNK), jnp.int32),
        pltpu.VMEM((2, CHUNK, D), jnp.float32),
        pltpu.VMEM_SHARED((NPAD, D), jnp.float32),
        pltpu.SemaphoreType.DMA,
        pltpu.SemaphoreType.DMA,
        pltpu.SemaphoreType.DMA,
        pltpu.SemaphoreType.DMA,
        pltpu.SemaphoreType.DMA,
    ],
)
def _spmm_k(y_hbm, src_hbm, dst_hbm, zero_hbm, out_hbm,
            dst_v, ring_v, rows_v, acc_sh, sg0, sg1, si0, si1, saux):
    c = lax.axis_index("c")
    s = lax.axis_index("s")
    w = c * NS + s
    sgs = (sg0, sg1)
    sis = (si0, si1)
    for b in range(2):
        pltpu.async_copy(src_hbm.at[w, b], ring_v.at[b], sis[b])
    pltpu.async_copy(dst_hbm.at[w], dst_v, saux)
    pltpu.sync_copy(zero_hbm, rows_v.at[0])
    for k in range(RPT // CHUNK):
        pltpu.async_copy(rows_v.at[0],
                         acc_sh.at[pl.ds(s * RPT + k * CHUNK, CHUNK)], saux)
    pltpu.make_async_copy(dst_hbm.at[w], dst_v, saux).wait()
    for k in range(RPT // CHUNK):
        pltpu.make_async_copy(
            rows_v.at[0], acc_sh.at[pl.ds(s * RPT, CHUNK)], saux).wait()
    plsc.subcore_barrier()

    for b in range(2):
        pltpu.make_async_copy(src_hbm.at[w, 0], ring_v.at[b], sis[b]).wait()
        pltpu.async_copy(y_hbm.at[ring_v.at[b]], rows_v.at[b], sgs[b])

    @pl.loop(0, NCHUNK, step=2)
    def _(j0):
        for b in range(2):
            j = j0 + b
            pltpu.make_async_copy(
                y_hbm.at[ring_v.at[b]], rows_v.at[b], sgs[b]).wait()

            @pl.when(j + 2 < NCHUNK)
            def _():
                pltpu.async_copy(src_hbm.at[w, j + 2], ring_v.at[b], sis[b])

            pltpu.sync_copy(rows_v.at[b], acc_sh.at[dst_v.at[j]], add=True)

            @pl.when(j + 2 < NCHUNK)
            def _():
                pltpu.make_async_copy(
                    src_hbm.at[w, 0], ring_v.at[b], sis[b]).wait()
                pltpu.async_copy(y_hbm.at[ring_v.at[b]], rows_v.at[b], sgs[b])

    plsc.subcore_barrier()
    pltpu.sync_copy(acc_sh.at[pl.ds(s * RPT, RPT)],
                    out_hbm.at[c, pl.ds(s * RPT, RPT)])


BLK = 2000


def _tc1(x, W1, deg_part_t):
    def body(x_ref, w_ref, dp_ref, y_ref, dinv_ref):
        deg = jnp.sum(dp_ref[...], axis=1) + 1.0
        dinv = lax.rsqrt(deg)[:, None]
        y_ref[...] = jnp.dot(x_ref[...], w_ref[...],
                             preferred_element_type=jnp.float32) * dinv
        dinv_ref[...] = dinv

    return pl.pallas_call(
        body,
        grid=(N // BLK,),
        in_specs=[
            pl.BlockSpec((BLK, D), lambda i: (i, 0)),
            pl.BlockSpec((D, D), lambda i: (0, 0)),
            pl.BlockSpec((BLK, NC), lambda i: (i, 0)),
        ],
        out_specs=[
            pl.BlockSpec((BLK, D), lambda i: (i, 0)),
            pl.BlockSpec((BLK, 1), lambda i: (i, 0)),
        ],
        out_shape=[
            jax.ShapeDtypeStruct((N, D), jnp.float32),
            jax.ShapeDtypeStruct((N, 1), jnp.float32),
        ],
    )(x, W1, deg_part_t)


def _tc2(p, y1, dinv, b1):
    def body(p_ref, y_ref, dinv_ref, b_ref, o_ref):
        t = p_ref[0] + p_ref[1] + y_ref[...]
        h = jnp.maximum(t * dinv_ref[...] + b_ref[...], 0.0)
        o_ref[...] = h * dinv_ref[...]

    return pl.pallas_call(
        body,
        grid=(N // BLK,),
        in_specs=[
            pl.BlockSpec((NC, BLK, D), lambda i: (0, i, 0)),
            pl.BlockSpec((BLK, D), lambda i: (i, 0)),
            pl.BlockSpec((BLK, 1), lambda i: (i, 0)),
            pl.BlockSpec((1, D), lambda i: (0, 0)),
        ],
        out_specs=pl.BlockSpec((BLK, D), lambda i: (i, 0)),
        out_shape=jax.ShapeDtypeStruct((N, D), jnp.float32),
    )(p, y1, dinv, b1)


def _tc3(q, y2, dinv, W2, b2, W3):
    def body(q_ref, y_ref, dinv_ref, w2_ref, b2_ref, w3_ref, o_ref):
        a1 = (q_ref[0] + q_ref[1] + y_ref[...]) * dinv_ref[...]
        h2 = jnp.maximum(
            jnp.dot(a1, w2_ref[...], preferred_element_type=jnp.float32)
            + b2_ref[...], 0.0)
        o_ref[...] = jnp.dot(h2, w3_ref[...],
                             preferred_element_type=jnp.float32) * dinv_ref[...]

    return pl.pallas_call(
        body,
        grid=(N // BLK,),
        in_specs=[
            pl.BlockSpec((NC, BLK, D), lambda i: (0, i, 0)),
            pl.BlockSpec((BLK, D), lambda i: (i, 0)),
            pl.BlockSpec((BLK, 1), lambda i: (i, 0)),
            pl.BlockSpec((D, 2 * D), lambda i: (0, 0)),
            pl.BlockSpec((1, 2 * D), lambda i: (0, 0)),
            pl.BlockSpec((2 * D, D), lambda i: (0, 0)),
        ],
        out_specs=pl.BlockSpec((BLK, D), lambda i: (i, 0)),
        out_shape=jax.ShapeDtypeStruct((N, D), jnp.float32),
    )(q, y2, dinv, W2, b2, W3)


def _tc4(r, y3, dinv, b3, batch2d):
    def body(r_ref, y_ref, dinv_ref, b_ref, batch_ref, o_ref, sums_sc, cnts_sc):
        j = pl.program_id(0)

        @pl.when(j == 0)
        def _():
            sums_sc[...] = jnp.zeros_like(sums_sc)
            cnts_sc[...] = jnp.zeros_like(cnts_sc)

        h3 = (r_ref[0] + r_ref[1] + y_ref[...]) * dinv_ref[...] + b_ref[...]
        gids = lax.broadcasted_iota(jnp.int32, (1, G), 1)
        M = (batch_ref[...] == gids).astype(jnp.float32)
        sums_sc[...] += lax.dot_general(
            M, h3, (((0,), (0,)), ((), ())),
            preferred_element_type=jnp.float32)
        cnts_sc[...] += lax.dot_general(
            M, jnp.ones((BLK, 1), jnp.float32), (((0,), (0,)), ((), ())),
            preferred_element_type=jnp.float32)

        @pl.when(j == pl.num_programs(0) - 1)
        def _():
            o_ref[...] = sums_sc[...] / jnp.maximum(cnts_sc[...], 1.0)

    return pl.pallas_call(
        body,
        grid=(N // BLK,),
        in_specs=[
            pl.BlockSpec((NC, BLK, D), lambda i: (0, i, 0)),
            pl.BlockSpec((BLK, D), lambda i: (i, 0)),
            pl.BlockSpec((BLK, 1), lambda i: (i, 0)),
            pl.BlockSpec((1, D), lambda i: (0, 0)),
            pl.BlockSpec((BLK, 1), lambda i: (i, 0)),
        ],
        out_specs=pl.BlockSpec((G, D), lambda i: (0, 0)),
        out_shape=jax.ShapeDtypeStruct((G, D), jnp.float32),
        scratch_shapes=[
            pltpu.VMEM((G, D), jnp.float32),
            pltpu.VMEM((G, 1), jnp.float32),
        ],
    )(r, y3, dinv, b3, batch2d)


def kernel(x, edge_index, batch, W1, b1, W2, b2, W3, b3):
    src = edge_index[0]
    dst = edge_index[1]
    pad = PE - E
    ar = jnp.arange(pad, dtype=jnp.int32)
    src_p = jnp.concatenate([src, ar % N]).reshape(NW, NCHUNK, CHUNK)
    dst_p = jnp.concatenate([dst, N + ar % (NPAD - N)]).reshape(NW, NCHUNK, CHUNK)

    deg_part = _deg_k(dst_p)
    y1, dinv = _tc1(x, W1, deg_part.T)

    zero_blk = jnp.zeros((CHUNK, D), jnp.float32)
    p = _spmm_k(y1, src_p, dst_p, zero_blk)
    y2 = _tc2(p, y1, dinv, b1.reshape(1, D))
    q = _spmm_k(y2, src_p, dst_p, zero_blk)
    y3 = _tc3(q, y2, dinv, W2, b2.reshape(1, 2 * D), W3)
    r = _spmm_k(y3, src_p, dst_p, zero_blk)
    return _tc4(r, y3, dinv, b3.reshape(1, D), batch.reshape(N, 1))
